# baseline (speedup 1.0000x reference)
import functools

import jax
import jax.numpy as jnp
from jax import lax
from jax.experimental import pallas as pl
from jax.experimental.pallas import tpu as pltpu

N_USERS = 50000
N_ITEMS = 25000
D = 64
LAYERS = 2
U1 = N_USERS + 1
I1 = N_ITEMS + 1
N = U1 + I1
TEMP = 0.2
LAMBDA_S = 0.5
NEG_EDGE = 4
CON = 0.1
GEN = 0.1
B = 1024


def _con_body(p_ref, a_ref, o_ref):
    p = p_ref[...]
    a = a_ref[...]
    pn = p / (jnp.sqrt(jnp.sum(p * p, axis=1, keepdims=True)) + 1e-12)
    an = a / (jnp.sqrt(jnp.sum(a * a, axis=1, keepdims=True)) + 1e-12)
    pos = jnp.sum(pn * an, axis=1) / TEMP
    m = lax.dot_general(pn, an, (((1,), (1,)), ((), ())),
                        preferred_element_type=jnp.float32) / TEMP
    ttl = jnp.sum(jnp.exp(m), axis=1)
    o_ref[...] = (-jnp.mean(pos - jnp.log(ttl))).reshape(1, 1)


def _con_loss_pallas(p, a):
    return pl.pallas_call(
        _con_body,
        out_shape=jax.ShapeDtypeStruct((1, 1), jnp.float32),
    )(p, a)[0, 0]


def _gen_body(pu_ref, pi_ref, nu_ref, ni_ref, o_ref):
    ps = jax.nn.sigmoid(jnp.sum(pu_ref[...] * pi_ref[...], axis=1))
    ns = jax.nn.sigmoid(jnp.sum(nu_ref[...] * ni_ref[...], axis=1))
    ps = jnp.clip(ps, 1e-7, 1.0 - 1e-7)
    ns = jnp.clip(ns, 1e-7, 1.0 - 1e-7)
    tot = jnp.sum(jnp.log(ps)) + jnp.sum(jnp.log(1.0 - ns))
    o_ref[...] = (-tot / (ps.shape[0] + ns.shape[0])).reshape(1, 1)


def _gen_loss_pallas(pu, pi, nu, ni):
    return pl.pallas_call(
        _gen_body,
        out_shape=jax.ShapeDtypeStruct((1, 1), jnp.float32),
    )(pu, pi, nu, ni)[0, 0]


def _bpr_body(ugu_ref, igp_ref, ign_ref, ulu_ref, ilp_ref, iln_ref, pair_ref,
              o_ref):
    sg_p = jnp.sum(ugu_ref[...] * igp_ref[...], axis=1)
    sg_n = jnp.sum(ugu_ref[...] * ign_ref[...], axis=1)
    sl_p = jnp.sum(ulu_ref[...] * ilp_ref[...], axis=1)
    sl_n = jnp.sum(ulu_ref[...] * iln_ref[...], axis=1)
    bp = LAMBDA_S * sg_p + (1.0 - LAMBDA_S) * sl_p
    bn = LAMBDA_S * sg_n + (1.0 - LAMBDA_S) * sl_n
    mask = jnp.any(pair_ref[...][:, :3] != 0, axis=1).astype(jnp.float32)
    per = -jnp.log(1e-10 + jax.nn.sigmoid(bp - bn))
    o_ref[...] = (jnp.sum(per * mask) /
                  jnp.maximum(jnp.sum(mask), 1.0)).reshape(1, 1)


def _bpr_pallas(ugu, igp, ign, ulu, ilp, iln, pair):
    return pl.pallas_call(
        _bpr_body,
        out_shape=jax.ShapeDtypeStruct((1, 1), jnp.float32),
    )(ugu, igp, ign, ulu, ilp, iln, pair)[0, 0]


def _lightgcn_xla(emb, ei):
    src, dst = ei[0].astype(jnp.int32), ei[1].astype(jnp.int32)
    deg = jnp.zeros((N,), jnp.float32).at[dst].add(1.0)
    deg = jnp.maximum(deg, 1.0)
    dinv = 1.0 / jnp.sqrt(deg)
    w = dinv[src] * dinv[dst]
    acc = emb
    h = emb
    for _ in range(LAYERS):
        h = jnp.zeros((N, emb.shape[1]), emb.dtype).at[dst].add(h[src] * w[:, None])
        acc = acc + h
    return acc / (LAYERS + 1)


def kernel(user_emb_glo, item_emb_glo, user_emb_loc, item_emb_loc,
           edge_index_view, edge_index_cart, edge_index_tar,
           edge_index_glo, edge_index_glo_aug,
           inter_view, inter_cart, inter_tar, batch_data):
    rk = jax.random.key(42)
    emb_loc = jnp.concatenate([user_emb_loc, item_emb_loc], axis=0)
    emb_glo = jnp.concatenate([user_emb_glo, item_emb_glo], axis=0)
    view_e = _lightgcn_xla(emb_loc, edge_index_view)
    cart_e = _lightgcn_xla(emb_loc, edge_index_cart)
    tar_e = _lightgcn_xla(emb_loc, edge_index_tar)
    glo_e = _lightgcn_xla(emb_glo, edge_index_glo)
    glo_a = _lightgcn_xla(emb_glo, edge_index_glo_aug)
    uv, iv = view_e[:U1], view_e[U1:]
    uc, ic = cart_e[:U1], cart_e[U1:]
    ut, it = tar_e[:U1], tar_e[U1:]
    ug, ig = glo_e[:U1], glo_e[U1:]
    uga, iga = glo_a[:U1], glo_a[U1:]

    def con(pos, aug, key):
        idx = jax.random.permutation(key, pos.shape[0])[:1024]
        return _con_loss_pallas(pos[idx], aug[idx])

    c_loss = (con(ug, uga, jax.random.fold_in(rk, 1)) +
              con(ig, iga, jax.random.fold_in(rk, 2))) / 2.0

    def gen(ue, ie, coo, key):
        k1, k2, k3 = jax.random.split(key, 3)
        samp = jax.random.randint(k1, (1024,), 0, coo.shape[1])
        pu = coo[0][samp].astype(jnp.int32)
        pi = coo[1][samp].astype(jnp.int32)
        nu = jax.random.randint(k2, (1024 * NEG_EDGE,), 0, ue.shape[0])
        ni = jax.random.randint(k3, (1024 * NEG_EDGE,), 0, ie.shape[0])
        return _gen_loss_pallas(ue[pu], ie[pi], ue[nu], ie[ni])

    bce_rv = (gen(uv, iv, inter_tar, jax.random.fold_in(rk, 3)) +
              gen(uc, ic, inter_view, jax.random.fold_in(rk, 4)) +
              gen(ut, it, inter_cart, jax.random.fold_in(rk, 5))) / 3.0
    bce_fw = (gen(uv, iv, inter_cart, jax.random.fold_in(rk, 6)) +
              gen(uc, ic, inter_tar, jax.random.fold_in(rk, 7)) +
              gen(ut, it, inter_view, jax.random.fold_in(rk, 8))) / 3.0
    bce_loss = (bce_rv + bce_fw) / 2.0

    u_loc = (uv + uc + ut) / 3.0
    i_loc = (iv + ic + it) / 3.0
    pair = batch_data[:, -1, :].astype(jnp.int32)
    us = pair[:, 0]
    itp = pair[:, 1]
    itn = pair[:, 2]
    bpr_loss = _bpr_pallas(ug[us], ig[itp], ig[itn],
                           u_loc[us], i_loc[itp], i_loc[itn], pair)

    return bpr_loss + CON * c_loss + GEN * bce_loss


# speedup vs baseline: 8.3884x; 8.3884x over previous
import functools

import jax
import jax.numpy as jnp
from jax import lax
from jax.experimental import pallas as pl
from jax.experimental.pallas import tpu as pltpu

N_USERS = 50000
N_ITEMS = 25000
D = 64
LAYERS = 2
U1 = N_USERS + 1
I1 = N_ITEMS + 1
N = U1 + I1
TEMP = 0.2
LAMBDA_S = 0.5
NEG_EDGE = 4
CON = 0.1
GEN = 0.1
B = 1024


def _con_body(p_ref, a_ref, o_ref):
    p = p_ref[...]
    a = a_ref[...]
    pn = p / (jnp.sqrt(jnp.sum(p * p, axis=1, keepdims=True)) + 1e-12)
    an = a / (jnp.sqrt(jnp.sum(a * a, axis=1, keepdims=True)) + 1e-12)
    pos = jnp.sum(pn * an, axis=1) / TEMP
    m = lax.dot_general(pn, an, (((1,), (1,)), ((), ())),
                        preferred_element_type=jnp.float32) / TEMP
    ttl = jnp.sum(jnp.exp(m), axis=1)
    o_ref[...] = (-jnp.mean(pos - jnp.log(ttl))).reshape(1, 1)


def _con_loss_pallas(p, a):
    return pl.pallas_call(
        _con_body,
        out_shape=jax.ShapeDtypeStruct((1, 1), jnp.float32),
    )(p, a)[0, 0]


def _gen_body(pu_ref, pi_ref, nu_ref, ni_ref, o_ref):
    ps = jax.nn.sigmoid(jnp.sum(pu_ref[...] * pi_ref[...], axis=1))
    ns = jax.nn.sigmoid(jnp.sum(nu_ref[...] * ni_ref[...], axis=1))
    ps = jnp.clip(ps, 1e-7, 1.0 - 1e-7)
    ns = jnp.clip(ns, 1e-7, 1.0 - 1e-7)
    tot = jnp.sum(jnp.log(ps)) + jnp.sum(jnp.log(1.0 - ns))
    o_ref[...] = (-tot / (ps.shape[0] + ns.shape[0])).reshape(1, 1)


def _gen_loss_pallas(pu, pi, nu, ni):
    return pl.pallas_call(
        _gen_body,
        out_shape=jax.ShapeDtypeStruct((1, 1), jnp.float32),
    )(pu, pi, nu, ni)[0, 0]


def _bpr_body(ugu_ref, igp_ref, ign_ref, ulu_ref, ilp_ref, iln_ref, pair_ref,
              o_ref):
    sg_p = jnp.sum(ugu_ref[...] * igp_ref[...], axis=1)
    sg_n = jnp.sum(ugu_ref[...] * ign_ref[...], axis=1)
    sl_p = jnp.sum(ulu_ref[...] * ilp_ref[...], axis=1)
    sl_n = jnp.sum(ulu_ref[...] * iln_ref[...], axis=1)
    bp = LAMBDA_S * sg_p + (1.0 - LAMBDA_S) * sl_p
    bn = LAMBDA_S * sg_n + (1.0 - LAMBDA_S) * sl_n
    mask = jnp.any(pair_ref[...][:, :3] != 0, axis=1).astype(jnp.float32)
    per = -jnp.log(1e-10 + jax.nn.sigmoid(bp - bn))
    o_ref[...] = (jnp.sum(per * mask) /
                  jnp.maximum(jnp.sum(mask), 1.0)).reshape(1, 1)


def _bpr_pallas(ugu, igp, ign, ulu, ilp, iln, pair):
    return pl.pallas_call(
        _bpr_body,
        out_shape=jax.ShapeDtypeStruct((1, 1), jnp.float32),
    )(ugu, igp, ign, ulu, ilp, iln, pair)[0, 0]


from jax.experimental.pallas import tpu_sc as plsc

NPAD = 75776
USEG = 25088
ISEG = 12544
ACC_ROWS = 25344
SENT = 75700
BLKE = 256
DSEG = NPAD // 16

_MESH = plsc.VectorSubcoreMesh(core_axis_name="c", subcore_axis_name="s")

_EHP = {400000: 204800, 1200000: 606208, 1080000: 540672}


def _prep_edges(ei):
    src = ei[0].astype(jnp.int32)
    dst = ei[1].astype(jnp.int32)
    eh = src.shape[0] // 2
    ehp = _EHP[src.shape[0]]
    pad = ehp - eh
    padsrc = jnp.arange(pad, dtype=jnp.int32) % 2048
    paddst = jnp.full((pad,), SENT, jnp.int32)
    return (jnp.concatenate([src[:eh], padsrc]),
            jnp.concatenate([dst[:eh], paddst]),
            jnp.concatenate([src[eh:], padsrc]),
            jnp.concatenate([dst[eh:], paddst]))


def _deg_body(*refs):
    d_hbm = refs[:10]
    ones_hbm, zeros_hbm = refs[10], refs[11]
    outs = refs[12:22]
    deg_sh, dstv, t0, t1, t2, ones_v = refs[22:]
    c = lax.axis_index("c")
    s = lax.axis_index("s")
    w = s * 2 + c
    pltpu.sync_copy(ones_hbm, ones_v)
    tails = {2304: t0, 2560: t1, 512: t2}
    for g in range(5):
        pltpu.sync_copy(zeros_hbm, deg_sh.at[pl.ds(s * DSEG, DSEG)])
        plsc.subcore_barrier()
        for e in (d_hbm[2 * g], d_hbm[2 * g + 1]):
            chunk = e.shape[0] // 32
            nfull, tail = chunk // 4096, chunk % 4096

            def blk(b, _, e=e, chunk=chunk):
                off = w * chunk + b * 4096
                pltpu.sync_copy(e.at[pl.ds(off, 4096)], dstv)
                pltpu.sync_copy(ones_v, deg_sh.at[dstv], add=True)
                return 0
            lax.fori_loop(0, nfull, blk, 0)
            if tail:
                tb = tails[tail]
                off = w * chunk + nfull * 4096
                pltpu.sync_copy(e.at[pl.ds(off, tail)], tb)
                pltpu.sync_copy(ones_v.at[pl.ds(0, tail)], deg_sh.at[tb],
                                add=True)
        plsc.subcore_barrier()

        @pl.when(c == 0)
        def _():
            pltpu.sync_copy(deg_sh.at[pl.ds(s * DSEG, DSEG)],
                            outs[2 * g].at[pl.ds(s * DSEG, DSEG)])

        @pl.when(c == 1)
        def _():
            pltpu.sync_copy(deg_sh.at[pl.ds(s * DSEG, DSEG)],
                            outs[2 * g + 1].at[pl.ds(s * DSEG, DSEG)])
        plsc.subcore_barrier()


def _deg_call(e_list, ones_hbm, zeros_hbm):
    f = pl.kernel(
        _deg_body,
        out_type=tuple(jax.ShapeDtypeStruct((NPAD,), jnp.float32)
                       for _ in range(10)),
        mesh=_MESH,
        scratch_types=[
            pltpu.VMEM_SHARED((NPAD,), jnp.float32),
            pltpu.VMEM((4096,), jnp.int32),
            pltpu.VMEM((2304,), jnp.int32),
            pltpu.VMEM((2560,), jnp.int32),
            pltpu.VMEM((512,), jnp.int32),
            pltpu.VMEM((4096,), jnp.float32),
        ],
    )
    dsts = []
    for e4 in e_list:
        dsts += [e4[1], e4[3]]
    return f(*dsts, ones_hbm, zeros_hbm)


def _prop_body(*refs):
    e_hbm = refs[:20]
    g_hbm = refs[20:25]
    zc_hbm = refs[25]
    outs = refs[26:36]
    acc_sh, zero_c, srcv, dstv, locv, rows_v = refs[36:]
    c = lax.axis_index("c")
    s = lax.axis_index("s")
    iota = lax.iota(jnp.int32, 16)
    pltpu.sync_copy(zc_hbm, zero_c)
    for g in range(5):
        gt = g_hbm[g]
        ehp = e_hbm[4 * g].shape[0]
        chunk = ehp // 16
        nblk = chunk // BLKE
        for half in (0, 1):
            if half == 0:
                base = U1 + c * ISEG
                r = ISEG
                zrows = 800
                p = outs[2 * g + 1]
                wout = 784
            else:
                base = c * USEG
                r = USEG
                zrows = 1584
                p = outs[2 * g]
                wout = 1568

            nzfull, zrem = zrows // 160, zrows % 160

            def zblk(i, _, zrows=zrows):
                pltpu.sync_copy(
                    zero_c, acc_sh.at[pl.ds(s * zrows + i * 160, 160), :])
                return 0
            lax.fori_loop(0, nzfull, zblk, 0)
            if zrem:
                pltpu.sync_copy(
                    zero_c.at[pl.ds(0, zrem), :],
                    acc_sh.at[pl.ds(s * zrows + nzfull * 160, zrem), :])
            plsc.subcore_barrier()

            se = e_hbm[4 * g + (0 if half == 0 else 2)]
            de = e_hbm[4 * g + (1 if half == 0 else 3)]

            def blk(b, _, se=se, de=de, gt=gt, base=base, r=r, chunk=chunk):
                off = s * chunk + b * BLKE
                pltpu.sync_copy(se.at[pl.ds(off, BLKE)], srcv)
                pltpu.sync_copy(de.at[pl.ds(off, BLKE)], dstv)
                dummy0 = r + (b & 15)

                def grp(j, _):
                    d16 = dstv[pl.ds(j * 16, 16)]
                    m = (d16 >= base) & (d16 < base + r)
                    loc = jnp.where(m, d16 - base, dummy0 + iota * 16)
                    locv[pl.ds(j * 16, 16)] = loc
                    return 0
                lax.fori_loop(0, BLKE // 16, grp, 0)
                pltpu.sync_copy(gt.at[srcv], rows_v)
                pltpu.sync_copy(rows_v, acc_sh.at[locv], add=True)
                return 0
            lax.fori_loop(0, nblk, blk, 0)
            plsc.subcore_barrier()

            pltpu.sync_copy(
                acc_sh.at[pl.ds(s * wout, wout), :],
                p.at[pl.ds(c * r + s * wout, wout), :])
            plsc.subcore_barrier()


def _prop_call(e_flat, g_list, zc_hbm):
    shapes = []
    for _ in range(5):
        shapes += [jax.ShapeDtypeStruct((2 * USEG, D), jnp.float32),
                   jax.ShapeDtypeStruct((2 * ISEG, D), jnp.float32)]
    f = pl.kernel(
        _prop_body,
        out_type=tuple(shapes),
        mesh=_MESH,
        scratch_types=[
            pltpu.VMEM_SHARED((ACC_ROWS, D), jnp.float32),
            pltpu.VMEM((160, D), jnp.float32),
            pltpu.VMEM((BLKE,), jnp.int32),
            pltpu.VMEM((BLKE,), jnp.int32),
            pltpu.VMEM((BLKE,), jnp.int32),
            pltpu.VMEM((BLKE, D), jnp.float32),
        ],
        compiler_params=pltpu.CompilerParams(use_tc_tiling_on_sc=False),
    )
    outs = f(*e_flat, *g_list, zc_hbm)
    ps = []
    for g in range(5):
        pu, pi = outs[2 * g], outs[2 * g + 1]
        ps.append(jnp.concatenate(
            [pu[:U1], pi[:I1], jnp.zeros((NPAD - N, D), jnp.float32)]))
    return ps


_RB = 2048
_GRID = NPAD // _RB


def _ew_call(body, n_out, *args):
    spec = pl.BlockSpec((_RB, D), lambda i: (i, 0))
    return pl.pallas_call(
        body,
        grid=(_GRID,),
        in_specs=[spec] * len(args),
        out_specs=[spec] * n_out if n_out > 1 else spec,
        out_shape=(tuple(jax.ShapeDtypeStruct((NPAD, D), jnp.float32)
                         for _ in range(n_out)) if n_out > 1
                   else jax.ShapeDtypeStruct((NPAD, D), jnp.float32)),
    )(*args)


def _rsqrt_body(da_ref, db_ref, o_ref):
    deg = jnp.maximum(da_ref[...] + db_ref[...], 1.0)
    o_ref[...] = lax.rsqrt(deg)


def _g0_body(emb_ref, dv_ref, o_ref):
    o_ref[...] = emb_ref[...] * dv_ref[...]


def _post1_body(p_ref, dv_ref, emb_ref, g1_ref, part_ref):
    dv = dv_ref[...]
    h = dv * p_ref[...]
    g1_ref[...] = dv * h
    part_ref[...] = emb_ref[...] + h


def _post2_body(p_ref, dv_ref, part_ref, o_ref):
    o_ref[...] = (part_ref[...] + dv_ref[...] * p_ref[...]) * (1.0 / 3.0)


def _dinv_bcast(dega, degb):
    dinv = pl.pallas_call(
        _rsqrt_body,
        out_shape=jax.ShapeDtypeStruct((1, NPAD), jnp.float32),
    )(dega.reshape(1, NPAD), degb.reshape(1, NPAD))
    return jnp.broadcast_to(dinv[0][:, None], (NPAD, D))


def _lightgcn_sc(embp_list, e_list, zc_hbm, ones_hbm, zeros_hbm):
    e_flat = [a for e4 in e_list for a in e4]
    degs = _deg_call(e_list, ones_hbm, zeros_hbm)
    dinvs = [_dinv_bcast(degs[2 * g], degs[2 * g + 1]) for g in range(5)]
    g0s = [_ew_call(_g0_body, 1, embp_list[g], dinvs[g]) for g in range(5)]
    p1s = _prop_call(e_flat, g0s, zc_hbm)
    g1s, parts = [], []
    for g in range(5):
        g1, part = _ew_call(_post1_body, 2, p1s[g], dinvs[g], embp_list[g])
        g1s.append(g1)
        parts.append(part)
    p2s = _prop_call(e_flat, g1s, zc_hbm)
    return [_ew_call(_post2_body, 1, p2s[g], dinvs[g], parts[g])
            for g in range(5)]


def kernel(user_emb_glo, item_emb_glo, user_emb_loc, item_emb_loc,
           edge_index_view, edge_index_cart, edge_index_tar,
           edge_index_glo, edge_index_glo_aug,
           inter_view, inter_cart, inter_tar, batch_data):
    rk = jax.random.key(42)
    emb_loc = jnp.concatenate([user_emb_loc, item_emb_loc], axis=0)
    emb_glo = jnp.concatenate([user_emb_glo, item_emb_glo], axis=0)
    embp_loc = jnp.pad(emb_loc, ((0, NPAD - N), (0, 0)))
    embp_glo = jnp.pad(emb_glo, ((0, NPAD - N), (0, 0)))
    e_list = [_prep_edges(e) for e in (edge_index_view, edge_index_cart,
                                       edge_index_tar, edge_index_glo,
                                       edge_index_glo_aug)]
    zc_hbm = jnp.zeros((160, D), jnp.float32)
    ones_hbm = jnp.ones((4096,), jnp.float32)
    zeros_hbm = jnp.zeros((DSEG,), jnp.float32)
    outs = _lightgcn_sc([embp_loc] * 3 + [embp_glo] * 2, e_list,
                        zc_hbm, ones_hbm, zeros_hbm)
    view_e = outs[0][:N]
    cart_e = outs[1][:N]
    tar_e = outs[2][:N]
    glo_e = outs[3][:N]
    glo_a = outs[4][:N]
    uv, iv = view_e[:U1], view_e[U1:]
    uc, ic = cart_e[:U1], cart_e[U1:]
    ut, it = tar_e[:U1], tar_e[U1:]
    ug, ig = glo_e[:U1], glo_e[U1:]
    uga, iga = glo_a[:U1], glo_a[U1:]

    def con(pos, aug, key):
        idx = jax.random.permutation(key, pos.shape[0])[:1024]
        return _con_loss_pallas(pos[idx], aug[idx])

    c_loss = (con(ug, uga, jax.random.fold_in(rk, 1)) +
              con(ig, iga, jax.random.fold_in(rk, 2))) / 2.0

    def gen(ue, ie, coo, key):
        k1, k2, k3 = jax.random.split(key, 3)
        samp = jax.random.randint(k1, (1024,), 0, coo.shape[1])
        pu = coo[0][samp].astype(jnp.int32)
        pi = coo[1][samp].astype(jnp.int32)
        nu = jax.random.randint(k2, (1024 * NEG_EDGE,), 0, ue.shape[0])
        ni = jax.random.randint(k3, (1024 * NEG_EDGE,), 0, ie.shape[0])
        return _gen_loss_pallas(ue[pu], ie[pi], ue[nu], ie[ni])

    bce_rv = (gen(uv, iv, inter_tar, jax.random.fold_in(rk, 3)) +
              gen(uc, ic, inter_view, jax.random.fold_in(rk, 4)) +
              gen(ut, it, inter_cart, jax.random.fold_in(rk, 5))) / 3.0
    bce_fw = (gen(uv, iv, inter_cart, jax.random.fold_in(rk, 6)) +
              gen(uc, ic, inter_tar, jax.random.fold_in(rk, 7)) +
              gen(ut, it, inter_view, jax.random.fold_in(rk, 8))) / 3.0
    bce_loss = (bce_rv + bce_fw) / 2.0

    u_loc = (uv + uc + ut) / 3.0
    i_loc = (iv + ic + it) / 3.0
    pair = batch_data[:, -1, :].astype(jnp.int32)
    us = pair[:, 0]
    itp = pair[:, 1]
    itn = pair[:, 2]
    bpr_loss = _bpr_pallas(ug[us], ig[itp], ig[itn],
                           u_loc[us], i_loc[itp], i_loc[itn], pair)

    return bpr_loss + CON * c_loss + GEN * bce_loss
